# baseline (speedup 1.0000x reference)
import jax
import jax.numpy as jnp
from jax.experimental import pallas as pl
from jax.experimental.pallas import tpu as pltpu


def _chamfer_body(nc, n, m, x1_ref, x2t_ref, d1_ref, i1_ref, d2_ref, i2_ref):
    i = pl.program_id(1)
    x1 = x1_ref[0]
    x2t = x2t_ref[0]

    sq1 = jnp.sum(x1 * x1, axis=1, keepdims=True)
    sq2 = jnp.sum(x2t * x2t, axis=0, keepdims=True)
    inner2 = jax.lax.dot_general(
        x1.astype(jnp.bfloat16), (x2t * -2.0).astype(jnp.bfloat16),
        (((1,), (0,)), ((), ())),
        preferred_element_type=jnp.float32)
    d = (sq1 + sq2) + inner2

    rmin = jnp.min(d, axis=1, keepdims=True)
    rarg = jnp.argmin(d, axis=1).astype(jnp.int32)[:, None]
    d1_ref[0] = rmin
    i1_ref[0] = rarg

    cmin = jnp.min(d, axis=0, keepdims=True)
    carg = jnp.argmin(d, axis=0).astype(jnp.int32)[None, :] + i * nc

    @pl.when(i == 0)
    def _():
        d2_ref[0] = cmin
        i2_ref[0] = carg

    @pl.when(i != 0)
    def _():
        prev_d = d2_ref[0]
        prev_i = i2_ref[0]
        better = cmin < prev_d
        d2_ref[0] = jnp.where(better, cmin, prev_d)
        i2_ref[0] = jnp.where(better, carg, prev_i)


def _chamfer_onedir(x1, x2t, nc):
    b, n, _ = x1.shape
    m = x2t.shape[2]
    n_chunks = n // nc

    def body(x1_ref, x2t_ref, d1_ref, i1_ref, d2_ref, i2_ref):
        _chamfer_body(nc, n, m, x1_ref, x2t_ref, d1_ref, i1_ref, d2_ref,
                      i2_ref)

    d1, i1, d2, i2 = pl.pallas_call(
        body,
        grid=(b, n_chunks),
        in_specs=[
            pl.BlockSpec((1, nc, 3), lambda bb, ii: (bb, ii, 0)),
            pl.BlockSpec((1, 3, m), lambda bb, ii: (bb, 0, 0)),
        ],
        out_specs=[
            pl.BlockSpec((1, nc, 1), lambda bb, ii: (bb, ii, 0)),
            pl.BlockSpec((1, nc, 1), lambda bb, ii: (bb, ii, 0)),
            pl.BlockSpec((1, 1, m), lambda bb, ii: (bb, 0, 0)),
            pl.BlockSpec((1, 1, m), lambda bb, ii: (bb, 0, 0)),
        ],
        out_shape=[
            jax.ShapeDtypeStruct((b, n, 1), jnp.float32),
            jax.ShapeDtypeStruct((b, n, 1), jnp.int32),
            jax.ShapeDtypeStruct((b, 1, m), jnp.float32),
            jax.ShapeDtypeStruct((b, 1, m), jnp.int32),
        ],
        compiler_params=pltpu.CompilerParams(
            dimension_semantics=("parallel", "arbitrary")),
    )(x1, x2t)
    return (d1.reshape(b, n), i1.reshape(b, n),
            d2.reshape(b, m), i2.reshape(b, m))


@jax.jit
def kernel(input1, input2):
    x2t = input2.transpose(0, 2, 1)
    dist1, idx1, dist2, idx2 = _chamfer_onedir(input1, x2t, 2048)
    return (dist1, dist2, idx1, idx2)


# speedup vs baseline: 1.3293x; 1.3293x over previous
import jax
import jax.numpy as jnp
from jax.experimental import pallas as pl
from jax.experimental.pallas import tpu as pltpu

_LANE = 128


def _chamfer_body(nc, n, m, x1_ref, x2t_ref, d1_ref, i1_ref, d2_ref, i2_ref):
    i = pl.program_id(1)
    x1 = x1_ref[0]
    x2t = x2t_ref[0]

    sq1 = jnp.sum(x1 * x1, axis=1, keepdims=True)
    sq2 = jnp.sum(x2t * x2t, axis=0, keepdims=True)
    inner2 = jax.lax.dot_general(
        x1.astype(jnp.bfloat16), (x2t * -2.0).astype(jnp.bfloat16),
        (((1,), (0,)), ((), ())),
        preferred_element_type=jnp.float32)
    d = (sq1 + sq2) + inner2

    val = d[:, 0:_LANE]
    jdx = jnp.zeros((nc, _LANE), jnp.int32)
    for j in range(1, m // _LANE):
        s = d[:, j * _LANE:(j + 1) * _LANE]
        mask = s < val
        val = jnp.where(mask, s, val)
        jdx = jnp.where(mask, j, jdx)
    fidx = jdx * _LANE + jax.lax.broadcasted_iota(
        jnp.int32, (nc, _LANE), 1)

    valt = val.T
    fidxt = fidx.T
    rmin = jnp.min(valt, axis=0, keepdims=True)
    rarg = jnp.min(jnp.where(valt == rmin, fidxt, m),
                   axis=0, keepdims=True)
    d1_ref[0, 0] = rmin
    i1_ref[0, 0] = rarg

    cmin = jnp.min(d, axis=0, keepdims=True)
    carg = jnp.argmin(d, axis=0).astype(jnp.int32)[None, :] + i * nc

    @pl.when(i == 0)
    def _():
        d2_ref[0] = cmin
        i2_ref[0] = carg

    @pl.when(i != 0)
    def _():
        prev_d = d2_ref[0]
        prev_i = i2_ref[0]
        better = cmin < prev_d
        d2_ref[0] = jnp.where(better, cmin, prev_d)
        i2_ref[0] = jnp.where(better, carg, prev_i)


def _chamfer(x1, x2t, nc):
    b, n, _ = x1.shape
    m = x2t.shape[2]
    n_chunks = n // nc

    def body(*refs):
        _chamfer_body(nc, n, m, *refs)

    d1, i1, d2, i2 = pl.pallas_call(
        body,
        grid=(b, n_chunks),
        in_specs=[
            pl.BlockSpec((1, nc, 3), lambda bb, ii: (bb, ii, 0)),
            pl.BlockSpec((1, 3, m), lambda bb, ii: (bb, 0, 0)),
        ],
        out_specs=[
            pl.BlockSpec((1, 1, 1, nc), lambda bb, ii: (bb, ii, 0, 0)),
            pl.BlockSpec((1, 1, 1, nc), lambda bb, ii: (bb, ii, 0, 0)),
            pl.BlockSpec((1, 1, m), lambda bb, ii: (bb, 0, 0)),
            pl.BlockSpec((1, 1, m), lambda bb, ii: (bb, 0, 0)),
        ],
        out_shape=[
            jax.ShapeDtypeStruct((b, n_chunks, 1, nc), jnp.float32),
            jax.ShapeDtypeStruct((b, n_chunks, 1, nc), jnp.int32),
            jax.ShapeDtypeStruct((b, 1, m), jnp.float32),
            jax.ShapeDtypeStruct((b, 1, m), jnp.int32),
        ],
        compiler_params=pltpu.CompilerParams(
            dimension_semantics=("parallel", "arbitrary")),
    )(x1, x2t)
    return (d1.reshape(b, n), i1.reshape(b, n),
            d2.reshape(b, m), i2.reshape(b, m))


@jax.jit
def kernel(input1, input2):
    x2t = input2.transpose(0, 2, 1)
    dist1, idx1, dist2, idx2 = _chamfer(input1, x2t, 1024)
    return (dist1, dist2, idx1, idx2)


# speedup vs baseline: 1.4844x; 1.1167x over previous
import jax
import jax.numpy as jnp
from jax.experimental import pallas as pl
from jax.experimental.pallas import tpu as pltpu

_LANE = 128


def _chamfer_body(nc, n, m, x1_ref, x2t_ref, d1_ref, i1_ref, d2_ref, i2_ref):
    i = pl.program_id(1)
    x1 = x1_ref[0]
    x2t = x2t_ref[0]

    sq1 = jnp.sum(x1 * x1, axis=1, keepdims=True)
    sq2 = jnp.sum(x2t * x2t, axis=0, keepdims=True)
    inner2 = jax.lax.dot_general(
        x1.astype(jnp.bfloat16), (x2t * -2.0).astype(jnp.bfloat16),
        (((1,), (0,)), ((), ())),
        preferred_element_type=jnp.float32)
    d = (sq1 + sq2) + inner2

    val = d[:, 0:_LANE]
    jdx = jnp.zeros((nc, _LANE), jnp.int32)
    for j in range(1, m // _LANE):
        s = d[:, j * _LANE:(j + 1) * _LANE]
        mask = s < val
        val = jnp.where(mask, s, val)
        jdx = jnp.where(mask, j, jdx)
    fidx = jdx * _LANE + jax.lax.broadcasted_iota(
        jnp.int32, (nc, _LANE), 1)

    valt = val.T
    fidxt = fidx.T
    rmin = jnp.min(valt, axis=0, keepdims=True)
    rarg = jnp.min(jnp.where(valt == rmin, fidxt, m),
                   axis=0, keepdims=True)
    d1_ref[0, 0] = rmin
    i1_ref[0, 0] = rarg

    rval = d[0:8, :]
    ridx = jnp.zeros((8, m), jnp.int32)
    for r in range(1, nc // 8):
        s = d[r * 8:(r + 1) * 8, :]
        rmask = s < rval
        rval = jnp.where(rmask, s, rval)
        ridx = jnp.where(rmask, r, ridx)
    rfidx = ridx * 8 + jax.lax.broadcasted_iota(jnp.int32, (8, m), 0)

    cmin = jnp.min(rval, axis=0, keepdims=True)
    carg = jnp.min(jnp.where(rval == cmin, rfidx, nc),
                   axis=0, keepdims=True) + i * nc

    @pl.when(i == 0)
    def _():
        d2_ref[0] = cmin
        i2_ref[0] = carg

    @pl.when(i != 0)
    def _():
        prev_d = d2_ref[0]
        prev_i = i2_ref[0]
        better = cmin < prev_d
        d2_ref[0] = jnp.where(better, cmin, prev_d)
        i2_ref[0] = jnp.where(better, carg, prev_i)


def _chamfer(x1, x2t, nc):
    b, n, _ = x1.shape
    m = x2t.shape[2]
    n_chunks = n // nc

    def body(*refs):
        _chamfer_body(nc, n, m, *refs)

    d1, i1, d2, i2 = pl.pallas_call(
        body,
        grid=(b, n_chunks),
        in_specs=[
            pl.BlockSpec((1, nc, 3), lambda bb, ii: (bb, ii, 0)),
            pl.BlockSpec((1, 3, m), lambda bb, ii: (bb, 0, 0)),
        ],
        out_specs=[
            pl.BlockSpec((1, 1, 1, nc), lambda bb, ii: (bb, ii, 0, 0)),
            pl.BlockSpec((1, 1, 1, nc), lambda bb, ii: (bb, ii, 0, 0)),
            pl.BlockSpec((1, 1, m), lambda bb, ii: (bb, 0, 0)),
            pl.BlockSpec((1, 1, m), lambda bb, ii: (bb, 0, 0)),
        ],
        out_shape=[
            jax.ShapeDtypeStruct((b, n_chunks, 1, nc), jnp.float32),
            jax.ShapeDtypeStruct((b, n_chunks, 1, nc), jnp.int32),
            jax.ShapeDtypeStruct((b, 1, m), jnp.float32),
            jax.ShapeDtypeStruct((b, 1, m), jnp.int32),
        ],
        compiler_params=pltpu.CompilerParams(
            dimension_semantics=("parallel", "arbitrary")),
    )(x1, x2t)
    return (d1.reshape(b, n), i1.reshape(b, n),
            d2.reshape(b, m), i2.reshape(b, m))


@jax.jit
def kernel(input1, input2):
    x2t = input2.transpose(0, 2, 1)
    dist1, idx1, dist2, idx2 = _chamfer(input1, x2t, 1024)
    return (dist1, dist2, idx1, idx2)


# speedup vs baseline: 1.5260x; 1.0280x over previous
import jax
import jax.numpy as jnp
from jax.experimental import pallas as pl
from jax.experimental.pallas import tpu as pltpu

_LANE = 128


def _chamfer_body(nc, n, m, x1_ref, x2t_ref, d1_ref, i1_ref, d2_ref, i2_ref):
    i = pl.program_id(1)
    x1 = x1_ref[0]
    x2t = x2t_ref[0]

    sq1 = jnp.sum(x1 * x1, axis=1, keepdims=True)
    sq2 = jnp.sum(x2t * x2t, axis=0, keepdims=True)
    inner2 = jax.lax.dot_general(
        x1.astype(jnp.bfloat16), (x2t * -2.0).astype(jnp.bfloat16),
        (((1,), (0,)), ((), ())),
        preferred_element_type=jnp.float32)
    d = (sq1 + sq2) + inner2

    val = d[:, 0:_LANE]
    jdx = jnp.zeros((nc, _LANE), jnp.int32)
    for j in range(1, m // _LANE):
        s = d[:, j * _LANE:(j + 1) * _LANE]
        mask = s < val
        val = jnp.where(mask, s, val)
        jdx = jnp.where(mask, j, jdx)
    fidx = jdx * _LANE + jax.lax.broadcasted_iota(
        jnp.int32, (nc, _LANE), 1)

    valt = val.T
    fidxt = fidx.T
    rmin = jnp.min(valt, axis=0, keepdims=True)
    rarg = jnp.min(jnp.where(valt == rmin, fidxt, m),
                   axis=0, keepdims=True)
    d1_ref[0, 0] = rmin
    i1_ref[0, 0] = rarg

    rval = d[0:8, :]
    ridx = jnp.zeros((8, m), jnp.int32)
    for r in range(1, nc // 8):
        s = d[r * 8:(r + 1) * 8, :]
        rmask = s < rval
        rval = jnp.where(rmask, s, rval)
        ridx = jnp.where(rmask, r, ridx)
    rfidx = ridx * 8 + jax.lax.broadcasted_iota(jnp.int32, (8, m), 0)

    cmin = jnp.min(rval, axis=0, keepdims=True)
    carg = jnp.min(jnp.where(rval == cmin, rfidx, nc),
                   axis=0, keepdims=True) + i * nc

    @pl.when(i == 0)
    def _():
        d2_ref[0] = cmin
        i2_ref[0] = carg

    @pl.when(i != 0)
    def _():
        prev_d = d2_ref[0]
        prev_i = i2_ref[0]
        better = cmin < prev_d
        d2_ref[0] = jnp.where(better, cmin, prev_d)
        i2_ref[0] = jnp.where(better, carg, prev_i)


def _chamfer(x1, x2t, nc):
    b, n, _ = x1.shape
    m = x2t.shape[2]
    n_chunks = n // nc

    def body(*refs):
        _chamfer_body(nc, n, m, *refs)

    d1, i1, d2, i2 = pl.pallas_call(
        body,
        grid=(b, n_chunks),
        in_specs=[
            pl.BlockSpec((1, nc, 3), lambda bb, ii: (bb, ii, 0)),
            pl.BlockSpec((1, 3, m), lambda bb, ii: (bb, 0, 0)),
        ],
        out_specs=[
            pl.BlockSpec((1, 1, 1, nc), lambda bb, ii: (bb, ii, 0, 0)),
            pl.BlockSpec((1, 1, 1, nc), lambda bb, ii: (bb, ii, 0, 0)),
            pl.BlockSpec((1, 1, m), lambda bb, ii: (bb, 0, 0)),
            pl.BlockSpec((1, 1, m), lambda bb, ii: (bb, 0, 0)),
        ],
        out_shape=[
            jax.ShapeDtypeStruct((b, n_chunks, 1, nc), jnp.float32),
            jax.ShapeDtypeStruct((b, n_chunks, 1, nc), jnp.int32),
            jax.ShapeDtypeStruct((b, 1, m), jnp.float32),
            jax.ShapeDtypeStruct((b, 1, m), jnp.int32),
        ],
        compiler_params=pltpu.CompilerParams(
            dimension_semantics=("parallel", "arbitrary")),
    )(x1, x2t)
    return (d1.reshape(b, n), i1.reshape(b, n),
            d2.reshape(b, m), i2.reshape(b, m))


@jax.jit
def kernel(input1, input2):
    x2t = input2.transpose(0, 2, 1)
    dist1, idx1, dist2, idx2 = _chamfer(input1, x2t, 2048)
    return (dist1, dist2, idx1, idx2)
